# baseline (speedup 1.0000x reference)
import functools

import jax
import jax.numpy as jnp
from jax.experimental import pallas as pl
from jax.experimental.pallas import tpu as pltpu


def _elu(v):
    return jnp.where(v > 0, v, jnp.exp(jnp.minimum(v, 0.0)) - 1.0)


def _bf16(v):
    return v.astype(jnp.bfloat16)


def _make_kernel(n, nhid, block_m, nbuf, nsub):
    nblk = n // block_m
    sub_m = block_m // nsub

    def fused_kernel(adj_ref, x_ref, w1_ref, b1_ref, w2_ref, b2_ref, o_ref,
                     buf_ref, sem_ref, xv_ref, xsem_ref, sa_ref, sb_ref):

        def sub_copies(step):
            blk = jax.lax.rem(step, nblk)
            slot = jax.lax.rem(step, nbuf)
            return [
                pltpu.make_async_copy(
                    adj_ref.at[pl.ds(blk * block_m + j * sub_m, sub_m), :],
                    buf_ref.at[slot, pl.ds(j * sub_m, sub_m)],
                    sem_ref.at[slot],
                )
                for j in range(nsub)
            ]

        def start_copy(step):
            for c in sub_copies(step):
                c.start()

        def wait_copy(step):
            for c in sub_copies(step):
                c.wait()

        x_copy = pltpu.make_async_copy(x_ref, xv_ref, xsem_ref)
        x_copy.start()
        for s in range(nbuf):
            start_copy(s)
        x_copy.wait()

        sa_ref[...] = jnp.dot(xv_ref[...], w1_ref[...],
                              preferred_element_type=jnp.float32)

        def body(step, carry):
            blk = jax.lax.rem(step, nblk)
            slot = jax.lax.rem(step, nbuf)
            wait_copy(step)
            a = buf_ref[slot]

            @pl.when(step < nblk)
            def _():
                acc = jnp.dot(a, sa_ref[...], preferred_element_type=jnp.float32)
                h = _elu(acc + b1_ref[...])
                sb_ref[pl.ds(blk * block_m, block_m), :] = jnp.dot(
                    h, w2_ref[...], preferred_element_type=jnp.float32)

            @pl.when(step >= nblk)
            def _():
                acc = jnp.dot(a, sb_ref[...],
                              preferred_element_type=jnp.float32)
                o_ref[pl.ds(blk * block_m, block_m), :] = _elu(acc + b2_ref[...])

            @pl.when(step + nbuf < 2 * nblk)
            def _():
                start_copy(step + nbuf)

            return carry

        jax.lax.fori_loop(0, 2 * nblk, body, 0)

    return fused_kernel


@functools.partial(jax.jit, static_argnames=("block_m", "nbuf", "nsub"))
def _forward(x, adj, W1, b1, W2, b2, block_m=200, nbuf=4, nsub=1):
    n, _ = x.shape
    nhid = W1.shape[1]

    return pl.pallas_call(
        _make_kernel(n, nhid, block_m, nbuf, nsub),
        in_specs=[
            pl.BlockSpec(memory_space=pl.ANY),
            pl.BlockSpec(memory_space=pl.ANY),
            pl.BlockSpec(memory_space=pltpu.MemorySpace.VMEM),
            pl.BlockSpec(memory_space=pltpu.MemorySpace.VMEM),
            pl.BlockSpec(memory_space=pltpu.MemorySpace.VMEM),
            pl.BlockSpec(memory_space=pltpu.MemorySpace.VMEM),
        ],
        out_specs=pl.BlockSpec(memory_space=pltpu.MemorySpace.VMEM),
        out_shape=jax.ShapeDtypeStruct((n, nhid), jnp.float32),
        scratch_shapes=[
            pltpu.VMEM((nbuf, block_m, n), jnp.float32),
            pltpu.SemaphoreType.DMA((nbuf,)),
            pltpu.VMEM(x.shape, jnp.float32),
            pltpu.SemaphoreType.DMA,
            pltpu.VMEM((n, nhid), jnp.float32),
            pltpu.VMEM((n, nhid), jnp.float32),
        ],
    )(adj, x, W1, b1.reshape(1, nhid), W2, b2.reshape(1, nhid))


def kernel(x, adj, W1, b1, W2, b2):
    return _forward(x, adj, W1, b1, W2, b2)


# speedup vs baseline: 1.0247x; 1.0247x over previous
import functools

import jax
import jax.numpy as jnp
from jax.experimental import pallas as pl
from jax.experimental.pallas import tpu as pltpu


def _elu(v):
    return jnp.where(v > 0, v, jnp.exp(jnp.minimum(v, 0.0)) - 1.0)


def _make_fused_kernel(block_m):
    def fused_kernel(adj_ref, x_ref, w1_ref, b1_ref, w2_ref, b2_ref, o_ref,
                     sa_ref, sb_ref):
        layer = pl.program_id(0)
        i = pl.program_id(1)

        @pl.when((layer == 0) & (i == 0))
        def _():
            sa_ref[...] = jnp.dot(x_ref[...], w1_ref[...],
                                  preferred_element_type=jnp.float32)

        @pl.when(layer == 0)
        def _():
            acc = jnp.dot(adj_ref[...], sa_ref[...],
                          preferred_element_type=jnp.float32)
            h = _elu(acc + b1_ref[...])
            sb_ref[pl.ds(i * block_m, block_m), :] = jnp.dot(
                h, w2_ref[...], preferred_element_type=jnp.float32)

        @pl.when(layer == 1)
        def _():
            acc = jnp.dot(adj_ref[...], sb_ref[...],
                          preferred_element_type=jnp.float32)
            o_ref[...] = _elu(acc + b2_ref[...])

    return fused_kernel


@functools.partial(jax.jit, static_argnames=("block_m",))
def _forward(x, adj, W1, b1, W2, b2, block_m=400):
    n, _ = x.shape
    nhid = W1.shape[1]

    return pl.pallas_call(
        _make_fused_kernel(block_m),
        grid=(2, n // block_m),
        in_specs=[
            pl.BlockSpec((block_m, n), lambda l, i: (i, 0)),
            pl.BlockSpec(x.shape, lambda l, i: (0, 0)),
            pl.BlockSpec(W1.shape, lambda l, i: (0, 0)),
            pl.BlockSpec((1, nhid), lambda l, i: (0, 0)),
            pl.BlockSpec(W2.shape, lambda l, i: (0, 0)),
            pl.BlockSpec((1, nhid), lambda l, i: (0, 0)),
        ],
        out_specs=pl.BlockSpec((block_m, nhid), lambda l, i: (l * i, 0)),
        out_shape=jax.ShapeDtypeStruct((n, nhid), jnp.float32),
        scratch_shapes=[
            pltpu.VMEM((n, nhid), jnp.float32),
            pltpu.VMEM((n, nhid), jnp.float32),
        ],
    )(adj, x, W1, b1.reshape(1, nhid), W2, b2.reshape(1, nhid))


def kernel(x, adj, W1, b1, W2, b2):
    return _forward(x, adj, W1, b1, W2, b2)
